# Initial kernel scaffold; baseline (speedup 1.0000x reference)
import jax
import jax.numpy as jnp
from jax.experimental import pallas as pl


def kernel(edge_index, emb_weight):
    raise NotImplementedError("write your pallas kernel here")


# speedup vs baseline: 3.8168x; 3.8168x over previous
import jax
import jax.numpy as jnp
from jax import lax
from jax.experimental import pallas as pl
from jax.experimental.pallas import tpu as pltpu, tpu_sc as plsc

EMBED_DIM = 32
NUM_EMB = 17
N_EDGES = 1600000
NUM_WORKERS = 32
EDGES_PER_WORKER = N_EDGES // NUM_WORKERS
CHUNK = 400
CHUNKS_PER_WORKER = EDGES_PER_WORKER // CHUNK
LANES = 16
GROUPS = CHUNK // LANES


def _sc_body(src_hbm, dst_hbm, table_hbm, out_hbm,
             table_v, src_v, dst_v, rows_v):
    wid = lax.axis_index("s") * 2 + lax.axis_index("c")
    pltpu.sync_copy(table_hbm, table_v)

    def chunk_body(i, carry):
        base = wid * EDGES_PER_WORKER + i * CHUNK
        pltpu.sync_copy(src_hbm.at[pl.ds(base, CHUNK)], src_v)
        pltpu.sync_copy(dst_hbm.at[pl.ds(base, CHUNK)], dst_v)

        def group_body(g, c2):
            s = src_v[pl.ds(g * LANES, LANES)]
            d = dst_v[pl.ds(g * LANES, LANES)]
            x = jnp.abs(s - d) - 1
            bits = plsc.bitcast(x.astype(jnp.float32), jnp.int32)
            code = jnp.where(x >= 1, (bits >> 23) - 126, 0)
            offv = code * EMBED_DIM
            for e in range(LANES):
                off = offv[e]
                lo = table_v[pl.ds(off, LANES)]
                hi = table_v[pl.ds(off + LANES, LANES)]
                row = g * LANES + e
                rows_v[row, pl.ds(0, LANES)] = lo
                rows_v[row, pl.ds(LANES, LANES)] = hi
            return c2
        lax.fori_loop(0, GROUPS, group_body, 0)

        pltpu.sync_copy(rows_v, out_hbm.at[pl.ds(base, CHUNK)])
        return carry

    lax.fori_loop(0, CHUNKS_PER_WORKER, chunk_body, 0)


def kernel(edge_index, emb_weight):
    mesh = plsc.VectorSubcoreMesh(core_axis_name="c", subcore_axis_name="s")
    run = pl.kernel(
        _sc_body,
        out_type=jax.ShapeDtypeStruct((N_EDGES, EMBED_DIM), jnp.float32),
        mesh=mesh,
        scratch_types=[
            pltpu.VMEM((NUM_EMB * EMBED_DIM,), jnp.float32),
            pltpu.VMEM((CHUNK,), jnp.int32),
            pltpu.VMEM((CHUNK,), jnp.int32),
            pltpu.VMEM((CHUNK, EMBED_DIM), jnp.float32),
        ],
        compiler_params=pltpu.CompilerParams(needs_layout_passes=False),
    )
    return run(edge_index[0], edge_index[1], emb_weight.reshape(-1))


# speedup vs baseline: 5.5634x; 1.4576x over previous
import jax
import jax.numpy as jnp
from jax import lax
from jax.experimental import pallas as pl
from jax.experimental.pallas import tpu as pltpu, tpu_sc as plsc

EMBED_DIM = 32
NUM_EMB = 17
N_EDGES = 1600000
NUM_WORKERS = 32
EDGES_PER_WORKER = N_EDGES // NUM_WORKERS
CHUNK = 400
CHUNKS_PER_WORKER = EDGES_PER_WORKER // CHUNK
PAIRS = CHUNKS_PER_WORKER // 2
LANES = 16
GROUPS = CHUNK // LANES


def _sc_body(src_hbm, dst_hbm, table_hbm, out_hbm,
             table_v, src0, dst0, src1, dst1, rows0, rows1,
             sem_in0, sem_in1, sem_out0, sem_out1):
    wid = lax.axis_index("s") * 2 + lax.axis_index("c")
    w0 = wid * EDGES_PER_WORKER
    pltpu.sync_copy(table_hbm, table_v)

    def fire_in(c, sv, dv, sem):
        base = w0 + c * CHUNK
        pltpu.async_copy(src_hbm.at[pl.ds(base, CHUNK)], sv, sem)
        pltpu.async_copy(dst_hbm.at[pl.ds(base, CHUNK)], dv, sem)

    def wait_in(c, sv, dv, sem):
        base = w0 + c * CHUNK
        pltpu.make_async_copy(src_hbm.at[pl.ds(base, CHUNK)], sv, sem).wait()
        pltpu.make_async_copy(dst_hbm.at[pl.ds(base, CHUNK)], dv, sem).wait()

    def fire_out(c, rows, sem):
        base = w0 + c * CHUNK
        pltpu.async_copy(rows, out_hbm.at[pl.ds(base, CHUNK)], sem)

    def wait_out(c, rows, sem):
        base = w0 + c * CHUNK
        pltpu.make_async_copy(rows, out_hbm.at[pl.ds(base, CHUNK)], sem).wait()

    def compute(sv, dv, rows):
        def group_body(g, c2):
            s = sv[pl.ds(g * LANES, LANES)]
            d = dv[pl.ds(g * LANES, LANES)]
            x = jnp.abs(s - d) - 1
            bits = plsc.bitcast(x.astype(jnp.float32), jnp.int32)
            code = jnp.where(x >= 1, (bits >> 23) - 126, 0)
            offv = code * EMBED_DIM
            for e in range(LANES):
                off = offv[e]
                lo = table_v[pl.ds(off, LANES)]
                hi = table_v[pl.ds(off + LANES, LANES)]
                row = g * LANES + e
                rows[row, pl.ds(0, LANES)] = lo
                rows[row, pl.ds(LANES, LANES)] = hi
            return c2
        lax.fori_loop(0, GROUPS, group_body, 0)

    fire_in(0, src0, dst0, sem_in0)
    fire_in(1, src1, dst1, sem_in1)

    def pair_body(k, carry):
        c0 = 2 * k
        wait_in(c0, src0, dst0, sem_in0)

        @pl.when(k > 0)
        def _():
            wait_out(c0 - 2, rows0, sem_out0)
        compute(src0, dst0, rows0)
        fire_out(c0, rows0, sem_out0)
        fire_in(c0 + 2, src0, dst0, sem_in0)

        c1 = 2 * k + 1
        wait_in(c1, src1, dst1, sem_in1)

        @pl.when(k > 0)
        def _():
            wait_out(c1 - 2, rows1, sem_out1)
        compute(src1, dst1, rows1)
        fire_out(c1, rows1, sem_out1)

        @pl.when(k < PAIRS - 1)
        def _():
            fire_in(c1 + 2, src1, dst1, sem_in1)
        return carry

    lax.fori_loop(0, PAIRS, pair_body, 0)

    last = CHUNKS_PER_WORKER - 1
    wait_in(last, src0, dst0, sem_in0)
    wait_out(last - 2, rows0, sem_out0)
    compute(src0, dst0, rows0)
    fire_out(last, rows0, sem_out0)
    wait_out(last - 1, rows1, sem_out1)
    wait_out(last, rows0, sem_out0)


def kernel(edge_index, emb_weight):
    mesh = plsc.VectorSubcoreMesh(core_axis_name="c", subcore_axis_name="s")
    run = pl.kernel(
        _sc_body,
        out_type=jax.ShapeDtypeStruct((N_EDGES, EMBED_DIM), jnp.float32),
        mesh=mesh,
        scratch_types=[
            pltpu.VMEM((NUM_EMB * EMBED_DIM,), jnp.float32),
            pltpu.VMEM((CHUNK,), jnp.int32),
            pltpu.VMEM((CHUNK,), jnp.int32),
            pltpu.VMEM((CHUNK,), jnp.int32),
            pltpu.VMEM((CHUNK,), jnp.int32),
            pltpu.VMEM((CHUNK, EMBED_DIM), jnp.float32),
            pltpu.VMEM((CHUNK, EMBED_DIM), jnp.float32),
            pltpu.SemaphoreType.DMA,
            pltpu.SemaphoreType.DMA,
            pltpu.SemaphoreType.DMA,
            pltpu.SemaphoreType.DMA,
        ],
        compiler_params=pltpu.CompilerParams(needs_layout_passes=False),
    )
    return run(edge_index[0], edge_index[1], emb_weight.reshape(-1))


# speedup vs baseline: 8.5459x; 1.5361x over previous
import jax
import jax.numpy as jnp
from jax import lax
from jax.experimental import pallas as pl
from jax.experimental.pallas import tpu as pltpu, tpu_sc as plsc

EMBED_DIM = 32
NUM_EMB = 17
N_EDGES = 1600000
NUM_WORKERS = 32
CHUNK = 1280
N_CHUNKS = N_EDGES // CHUNK
BASE_CHUNKS = N_CHUNKS // NUM_WORKERS
EXTRA = N_CHUNKS - BASE_CHUNKS * NUM_WORKERS
MAX_PAIRS = (BASE_CHUNKS + 2) // 2
LANES = 16
GROUPS = CHUNK // LANES


def _sc_body(src_hbm, dst_hbm, table_hbm, out_hbm,
             table_v, src0, dst0, src1, dst1, rows0, rows1,
             sem_in0, sem_in1, sem_out0, sem_out1):
    wid = lax.axis_index("s") * 2 + lax.axis_index("c")
    n_w = jnp.where(wid < EXTRA, BASE_CHUNKS + 1, BASE_CHUNKS)
    pltpu.sync_copy(table_hbm, table_v)

    def fire_in(i, sv, dv, sem):
        base = (wid + i * NUM_WORKERS) * CHUNK
        pltpu.async_copy(src_hbm.at[pl.ds(base, CHUNK)], sv, sem)
        pltpu.async_copy(dst_hbm.at[pl.ds(base, CHUNK)], dv, sem)

    def wait_in(i, sv, dv, sem):
        base = (wid + i * NUM_WORKERS) * CHUNK
        pltpu.make_async_copy(src_hbm.at[pl.ds(base, CHUNK)], sv, sem).wait()
        pltpu.make_async_copy(dst_hbm.at[pl.ds(base, CHUNK)], dv, sem).wait()

    def fire_out(i, rows, sem):
        base = (wid + i * NUM_WORKERS) * CHUNK
        pltpu.async_copy(rows, out_hbm.at[:, pl.ds(base, CHUNK)], sem)

    def wait_out(i, rows, sem):
        base = (wid + i * NUM_WORKERS) * CHUNK
        pltpu.make_async_copy(rows, out_hbm.at[:, pl.ds(base, CHUNK)],
                              sem).wait()

    def compute(sv, dv, rows):
        def group_body(g, c2):
            s = sv[pl.ds(g * LANES, LANES)]
            d = dv[pl.ds(g * LANES, LANES)]
            x = jnp.abs(s - d) - 1
            bits = plsc.bitcast(x.astype(jnp.float32), jnp.int32)
            code = jnp.where(x >= 1, (bits >> 23) - 126, 0)
            for c in range(EMBED_DIM):
                v = plsc.load_gather(table_v, [code + (c * NUM_EMB)])
                rows[c, pl.ds(g * LANES, LANES)] = v
            return c2
        lax.fori_loop(0, GROUPS, group_body, 0)

    fire_in(0, src0, dst0, sem_in0)
    fire_in(1, src1, dst1, sem_in1)

    def pair_body(k, carry):
        i0 = 2 * k
        wait_in(i0, src0, dst0, sem_in0)

        @pl.when(k > 0)
        def _():
            wait_out(i0 - 2, rows0, sem_out0)
        compute(src0, dst0, rows0)
        fire_out(i0, rows0, sem_out0)

        @pl.when(i0 + 2 < n_w)
        def _():
            fire_in(i0 + 2, src0, dst0, sem_in0)

        i1 = 2 * k + 1

        @pl.when(k > 0)
        def _():
            wait_out(i1 - 2, rows1, sem_out1)

        @pl.when(i1 < n_w)
        def _():
            wait_in(i1, src1, dst1, sem_in1)
            compute(src1, dst1, rows1)
            fire_out(i1, rows1, sem_out1)

            @pl.when(i1 + 2 < n_w)
            def _():
                fire_in(i1 + 2, src1, dst1, sem_in1)
        return carry

    lax.fori_loop(0, MAX_PAIRS, pair_body, 0)

    wait_out(n_w - 1, rows0, sem_out0)

    @pl.when(n_w == BASE_CHUNKS + 1)
    def _():
        wait_out(n_w - 1, rows1, sem_out1)


def kernel(edge_index, emb_weight):
    mesh = plsc.VectorSubcoreMesh(core_axis_name="c", subcore_axis_name="s")
    run = pl.kernel(
        _sc_body,
        out_type=jax.ShapeDtypeStruct((EMBED_DIM, N_EDGES), jnp.float32),
        mesh=mesh,
        scratch_types=[
            pltpu.VMEM((NUM_EMB * EMBED_DIM,), jnp.float32),
            pltpu.VMEM((CHUNK,), jnp.int32),
            pltpu.VMEM((CHUNK,), jnp.int32),
            pltpu.VMEM((CHUNK,), jnp.int32),
            pltpu.VMEM((CHUNK,), jnp.int32),
            pltpu.VMEM((EMBED_DIM, CHUNK), jnp.float32),
            pltpu.VMEM((EMBED_DIM, CHUNK), jnp.float32),
            pltpu.SemaphoreType.DMA,
            pltpu.SemaphoreType.DMA,
            pltpu.SemaphoreType.DMA,
            pltpu.SemaphoreType.DMA,
        ],
        compiler_params=pltpu.CompilerParams(needs_layout_passes=False),
    )
    out_t = run(edge_index[0], edge_index[1], emb_weight.T.reshape(-1))
    return out_t.T


# speedup vs baseline: 11.1186x; 1.3010x over previous
import jax
import jax.numpy as jnp
from jax import lax
from jax.experimental import pallas as pl
from jax.experimental.pallas import tpu as pltpu, tpu_sc as plsc

EMBED_DIM = 32
NUM_EMB = 17
N_EDGES = 1600000
NUM_WORKERS = 32
CHUNK = 1280
N_CHUNKS = N_EDGES // CHUNK
BASE_CHUNKS = N_CHUNKS // NUM_WORKERS
EXTRA = N_CHUNKS - BASE_CHUNKS * NUM_WORKERS
MAX_PAIRS = (BASE_CHUNKS + 2) // 2
LANES = 16
GROUPS = CHUNK // LANES


def _sc_body(edge_hbm, table_hbm, out_hbm,
             table_v, src0, dst0, src1, dst1, rows0, rows1,
             sem_in0, sem_in1, sem_out0, sem_out1):
    wid = lax.axis_index("s") * 2 + lax.axis_index("c")
    n_w = jnp.where(wid < EXTRA, BASE_CHUNKS + 1, BASE_CHUNKS)
    pltpu.sync_copy(table_hbm, table_v)

    def fire_in(i, sv, dv, sem):
        base = (wid + i * NUM_WORKERS) * CHUNK
        pltpu.async_copy(edge_hbm.at[0, pl.ds(base, CHUNK)], sv, sem)
        pltpu.async_copy(edge_hbm.at[1, pl.ds(base, CHUNK)], dv, sem)

    def wait_in(i, sv, dv, sem):
        base = (wid + i * NUM_WORKERS) * CHUNK
        pltpu.make_async_copy(edge_hbm.at[0, pl.ds(base, CHUNK)], sv,
                              sem).wait()
        pltpu.make_async_copy(edge_hbm.at[1, pl.ds(base, CHUNK)], dv,
                              sem).wait()

    def fire_out(i, rows, sem):
        base = (wid + i * NUM_WORKERS) * CHUNK
        pltpu.async_copy(rows, out_hbm.at[:, pl.ds(base, CHUNK)], sem)

    def wait_out(i, rows, sem):
        base = (wid + i * NUM_WORKERS) * CHUNK
        pltpu.make_async_copy(rows, out_hbm.at[:, pl.ds(base, CHUNK)],
                              sem).wait()

    def compute(sv, dv, rows):
        def group_body(g, c2):
            s = sv[pl.ds(g * LANES, LANES)]
            d = dv[pl.ds(g * LANES, LANES)]
            x = jnp.abs(s - d) - 1
            bits = plsc.bitcast(x.astype(jnp.float32), jnp.int32)
            code = jnp.where(x >= 1, (bits >> 23) - 126, 0)
            for c in range(EMBED_DIM):
                v = plsc.load_gather(table_v, [code + (c * NUM_EMB)])
                rows[c, pl.ds(g * LANES, LANES)] = v
            return c2
        lax.fori_loop(0, GROUPS, group_body, 0, unroll=4)

    fire_in(0, src0, dst0, sem_in0)
    fire_in(1, src1, dst1, sem_in1)

    def pair_body(k, carry):
        i0 = 2 * k
        wait_in(i0, src0, dst0, sem_in0)

        @pl.when(k > 0)
        def _():
            wait_out(i0 - 2, rows0, sem_out0)
        compute(src0, dst0, rows0)
        fire_out(i0, rows0, sem_out0)

        @pl.when(i0 + 2 < n_w)
        def _():
            fire_in(i0 + 2, src0, dst0, sem_in0)

        i1 = 2 * k + 1

        @pl.when(k > 0)
        def _():
            wait_out(i1 - 2, rows1, sem_out1)

        @pl.when(i1 < n_w)
        def _():
            wait_in(i1, src1, dst1, sem_in1)
            compute(src1, dst1, rows1)
            fire_out(i1, rows1, sem_out1)

            @pl.when(i1 + 2 < n_w)
            def _():
                fire_in(i1 + 2, src1, dst1, sem_in1)
        return carry

    lax.fori_loop(0, MAX_PAIRS, pair_body, 0)

    wait_out(n_w - 1, rows0, sem_out0)

    @pl.when(n_w == BASE_CHUNKS + 1)
    def _():
        wait_out(n_w - 1, rows1, sem_out1)


def kernel(edge_index, emb_weight):
    mesh = plsc.VectorSubcoreMesh(core_axis_name="c", subcore_axis_name="s")
    run = pl.kernel(
        _sc_body,
        out_type=jax.ShapeDtypeStruct((EMBED_DIM, N_EDGES), jnp.float32),
        mesh=mesh,
        scratch_types=[
            pltpu.VMEM((NUM_EMB * EMBED_DIM,), jnp.float32),
            pltpu.VMEM((CHUNK,), jnp.int32),
            pltpu.VMEM((CHUNK,), jnp.int32),
            pltpu.VMEM((CHUNK,), jnp.int32),
            pltpu.VMEM((CHUNK,), jnp.int32),
            pltpu.VMEM((EMBED_DIM, CHUNK), jnp.float32),
            pltpu.VMEM((EMBED_DIM, CHUNK), jnp.float32),
            pltpu.SemaphoreType.DMA,
            pltpu.SemaphoreType.DMA,
            pltpu.SemaphoreType.DMA,
            pltpu.SemaphoreType.DMA,
        ],
        compiler_params=pltpu.CompilerParams(needs_layout_passes=False),
    )
    out_t = run(edge_index, emb_weight.T.reshape(-1))
    return out_t.T
